# baseline (speedup 1.0000x reference)
import jax
import jax.numpy as jnp
from jax import lax
from jax.experimental import pallas as pl
from jax.experimental.pallas import tpu as pltpu

_BB = 2


def _vq_body(x_ref, cb_ref, onehot_ref, codebooks_ref, csq_ref):
    g = pl.program_id(0)
    cb = cb_ref[...]
    S, d = cb.shape
    L = x_ref.shape[1]

    @pl.when(g == 0)
    def _init_csq():
        csq_ref[...] = jnp.broadcast_to(
            jnp.sum(cb * cb, axis=1)[None, :], csq_ref.shape)

    for i in range(_BB):
        x = x_ref[i]
        cross = lax.dot_general(
            x, cb, (((1,), (1,)), ((), ())),
            preferred_element_type=jnp.float32)
        x_sq = jnp.sum(x * x, axis=1, keepdims=True)
        d2 = x_sq - 2.0 * cross + csq_ref[0:1, :]
        deltas = jnp.argmin(d2, axis=1).astype(jnp.int32)
        col = lax.broadcasted_iota(jnp.int32, (L, S), 1)
        onehot = (col == deltas[:, None]).astype(jnp.float32)
        onehot_ref[i] = onehot

        counts = jnp.sum(onehot, axis=0)
        sums = lax.dot_general(
            onehot, x, (((0,), (0,)), ((), ())),
            preferred_element_type=jnp.float32)
        c = counts[:, None]
        codebooks_ref[i] = jnp.where(
            c > 0.0, sums / jnp.maximum(c, 1.0), cb)


def kernel(x, codebook):
    B, L, d = x.shape
    S = codebook.shape[0]
    G = B // _BB
    onehot, codebooks = pl.pallas_call(
        _vq_body,
        grid=(G,),
        in_specs=[
            pl.BlockSpec((_BB, L, d), lambda g: (g, 0, 0)),
            pl.BlockSpec((S, d), lambda g: (0, 0)),
        ],
        out_specs=[
            pl.BlockSpec((_BB, L, S), lambda g: (g, 0, 0)),
            pl.BlockSpec((_BB, S, d), lambda g: (g, 0, 0)),
        ],
        out_shape=[
            jax.ShapeDtypeStruct((B, L, S), jnp.float32),
            jax.ShapeDtypeStruct((B, S, d), jnp.float32),
        ],
        scratch_shapes=[
            pltpu.VMEM((8, S), jnp.float32),
        ],
    )(x, codebook)
    return onehot, codebooks


# speedup vs baseline: 1.0319x; 1.0319x over previous
import jax
import jax.numpy as jnp
from jax import lax
from jax.experimental import pallas as pl


def _vq_body(x_ref, cb_ref, onehot_ref, codebooks_ref):
    x = x_ref[0]
    cb = cb_ref[...]
    L = x.shape[0]
    S = cb.shape[0]
    cross = lax.dot_general(
        x, cb, (((1,), (1,)), ((), ())),
        preferred_element_type=jnp.float32)
    x_sq = jnp.sum(x * x, axis=1, keepdims=True)
    c_sq = jnp.sum(cb * cb, axis=1)[None, :]
    d2 = x_sq - 2.0 * cross + c_sq
    deltas = jnp.argmin(d2, axis=1).astype(jnp.int32)
    col = lax.broadcasted_iota(jnp.int32, (L, S), 1)
    onehot = (col == deltas[:, None]).astype(jnp.float32)
    onehot_ref[0] = onehot

    counts = jnp.sum(onehot, axis=0)
    sums = lax.dot_general(
        onehot, x, (((0,), (0,)), ((), ())),
        preferred_element_type=jnp.float32)
    c = counts[:, None]
    codebooks_ref[0] = jnp.where(c > 0.0, sums / jnp.maximum(c, 1.0), cb)


def kernel(x, codebook):
    B, L, d = x.shape
    S = codebook.shape[0]
    onehot, codebooks = pl.pallas_call(
        _vq_body,
        grid=(B,),
        in_specs=[
            pl.BlockSpec((1, L, d), lambda b: (b, 0, 0)),
            pl.BlockSpec((S, d), lambda b: (0, 0)),
        ],
        out_specs=[
            pl.BlockSpec((1, L, S), lambda b: (b, 0, 0)),
            pl.BlockSpec((1, S, d), lambda b: (b, 0, 0)),
        ],
        out_shape=[
            jax.ShapeDtypeStruct((B, L, S), jnp.float32),
            jax.ShapeDtypeStruct((B, S, d), jnp.float32),
        ],
    )(x, codebook)
    return onehot, codebooks
